# baseline (speedup 1.0000x reference)
import jax
import jax.numpy as jnp
from jax.experimental import pallas as pl
from jax.experimental.pallas import tpu as pltpu

_C = 8
_THR = 0.8


def _half_seg(x_ref, w, lab):
    n = x_ref.shape[1]
    x = x_ref[0].astype(jnp.bfloat16)
    proj = jax.lax.dot_general(
        x, w, (((1,), (0,)), ((), ())),
        preferred_element_type=jnp.float32)
    ones = jnp.ones((n, 1), jnp.float32)
    proj_aug = jnp.concatenate([proj, ones], axis=1)
    cid = jax.lax.broadcasted_iota(jnp.int32, (_C, n), 0)
    oh = (jnp.broadcast_to(lab, (_C, n)) == cid).astype(jnp.bfloat16)
    return jax.lax.dot_general(
        oh, proj_aug.astype(jnp.bfloat16), (((1,), (0,)), ((), ())),
        preferred_element_type=jnp.float32)


def _tc_body(lab_ref, w_ref, bias_ref, xa_ref, xb_ref, feats_ref, scores_ref,
             s0_ref, s1_ref, cn_ref):
    b = pl.program_id(0)
    nb = pl.num_programs(0)
    nh = xa_ref.shape[1]

    w = w_ref[...].astype(jnp.bfloat16)
    seg = (_half_seg(xa_ref, w, lab_ref[pl.ds(b, 1), pl.ds(0, nh)])
           + _half_seg(xb_ref, w, lab_ref[pl.ds(b, 1), pl.ds(nh, nh)]))

    eye = (jax.lax.broadcasted_iota(jnp.int32, (_C, _C), 0)
           == jax.lax.broadcasted_iota(jnp.int32, (_C, _C), 1)
           ).astype(jnp.float32)
    s0_ref[pl.ds(b, 1), :] = jnp.sum(seg[:, 0:1] * eye, axis=0, keepdims=True)
    s1_ref[pl.ds(b, 1), :] = jnp.sum(seg[:, 1:2] * eye, axis=0, keepdims=True)
    cn_ref[pl.ds(b, 1), :] = jnp.sum(seg[:, 2:3] * eye, axis=0, keepdims=True)

    @pl.when(b == nb - 1)
    def _epilogue():
        nb_ = feats_ref.shape[0]
        cnt = jnp.maximum(cn_ref[...], 1.0)
        l0 = s0_ref[...] / cnt + bias_ref[0, 0]
        l1 = s1_ref[...] / cnt + bias_ref[0, 1]
        m = jnp.maximum(l0, l1)
        e0 = jnp.exp(l0 - m)
        e1 = jnp.exp(l1 - m)
        sc = e1 / (e0 + e1)
        lane = jax.lax.broadcasted_iota(jnp.int32, (nb_, _C), 1)
        mx = jnp.max(sc, axis=1, keepdims=True)
        mn = jnp.min(sc, axis=1, keepdims=True)
        idx_max = jnp.min(jnp.where(sc == mx, lane, _C), axis=1, keepdims=True)
        idx_min = jnp.min(jnp.where(sc == mn, lane, _C), axis=1, keepdims=True)
        sel = jnp.where(mx < _THR, idx_min, idx_max)
        selh = (lane == sel).astype(jnp.float32)
        f0 = jnp.sum(l0 * selh, axis=1, keepdims=True)
        f1 = jnp.sum(l1 * selh, axis=1, keepdims=True)
        feats_ref[...] = jnp.concatenate([f0, f1], axis=1)
        scores_ref[...] = sc


def kernel(inst_feat, cluster_labels, W_head, b_head):
    B, N, D = inst_feat.shape
    ncls = W_head.shape[1]
    bias = b_head.reshape(1, ncls)
    feats, scores = pl.pallas_call(
        _tc_body,
        grid=(B,),
        in_specs=[
            pl.BlockSpec((B, N), lambda b: (0, 0)),
            pl.BlockSpec((D, ncls), lambda b: (0, 0)),
            pl.BlockSpec((1, ncls), lambda b: (0, 0)),
            pl.BlockSpec((1, N // 2, D), lambda b: (b, 0, 0)),
            pl.BlockSpec((1, N // 2, D), lambda b: (b, 1, 0)),
        ],
        out_specs=[
            pl.BlockSpec((B, ncls), lambda b: (0, 0)),
            pl.BlockSpec((B, _C), lambda b: (0, 0)),
        ],
        out_shape=[
            jax.ShapeDtypeStruct((B, ncls), jnp.float32),
            jax.ShapeDtypeStruct((B, _C), jnp.float32),
        ],
        scratch_shapes=[
            pltpu.VMEM((B, _C), jnp.float32),
            pltpu.VMEM((B, _C), jnp.float32),
            pltpu.VMEM((B, _C), jnp.float32),
        ],
    )(cluster_labels, W_head, bias, inst_feat, inst_feat)
    return feats, scores
